# initial kernel scaffold (unmeasured)
import jax
import jax.numpy as jnp
from jax import lax
from jax.experimental import pallas as pl
from jax.experimental.pallas import tpu as pltpu


def kernel(
    x,
):
    def body(*refs):
        pass

    out_shape = jax.ShapeDtypeStruct(..., jnp.float32)
    return pl.pallas_call(body, out_shape=out_shape)(...)


# baseline (device time: 82729 ns/iter reference)
import jax
import jax.numpy as jnp
from jax import lax
from jax.experimental import pallas as pl
from jax.experimental.pallas import tpu as pltpu

N_Z = 4


def kernel(x):
    m_per, n_total = x.shape
    n_per = n_total // N_Z
    xbf = x.astype(jnp.bfloat16)

    def body(x_ref, out_ref, send_sems, recv_sems):
        my_x = lax.axis_index("x")
        my_y = lax.axis_index("y")
        my_z = lax.axis_index("z")

        barrier_sem = pltpu.get_barrier_semaphore()
        for dz in range(1, N_Z):
            pl.semaphore_signal(
                barrier_sem, inc=1,
                device_id=(my_x, my_y, (my_z + dz) % N_Z),
                device_id_type=pl.DeviceIdType.MESH,
            )
        pl.semaphore_wait(barrier_sem, N_Z - 1)

        sends = []
        for dz in range(1, N_Z):
            tz = (my_z + dz) % N_Z
            rdma = pltpu.make_async_remote_copy(
                src_ref=x_ref.at[:, pl.ds(tz * n_per, n_per)],
                dst_ref=out_ref.at[pl.ds(my_z * m_per, m_per), :],
                send_sem=send_sems.at[dz],
                recv_sem=recv_sems.at[dz],
                device_id=(my_x, my_y, tz),
                device_id_type=pl.DeviceIdType.MESH,
            )
            rdma.start()
            sends.append(rdma)

        out_ref[pl.ds(my_z * m_per, m_per), :] = x_ref[
            :, pl.ds(my_z * n_per, n_per)
        ]

        for dz in range(1, N_Z):
            sz = (my_z - dz) % N_Z
            recv = pltpu.make_async_remote_copy(
                src_ref=x_ref.at[:, pl.ds(sz * n_per, n_per)],
                dst_ref=out_ref.at[pl.ds(sz * m_per, m_per), :],
                send_sem=send_sems.at[dz],
                recv_sem=recv_sems.at[dz],
                device_id=(my_x, my_y, sz),
                device_id_type=pl.DeviceIdType.MESH,
            )
            recv.wait_recv()

        for rdma in sends:
            rdma.wait_send()

    out_shape = jax.ShapeDtypeStruct((N_Z * m_per, n_per), jnp.bfloat16)
    return pl.pallas_call(
        body,
        out_shape=out_shape,
        in_specs=[pl.BlockSpec(memory_space=pltpu.VMEM)],
        out_specs=pl.BlockSpec(memory_space=pltpu.VMEM),
        scratch_shapes=[
            pltpu.SemaphoreType.DMA((N_Z,)),
            pltpu.SemaphoreType.DMA((N_Z,)),
        ],
        compiler_params=pltpu.CompilerParams(collective_id=0),
    )(xbf)


# device time: 78446 ns/iter; 1.0546x vs baseline; 1.0546x over previous
import jax
import jax.numpy as jnp
from jax import lax
from jax.experimental import pallas as pl
from jax.experimental.pallas import tpu as pltpu

N_Z = 4


def kernel(x):
    m_per, n_total = x.shape
    n_per = n_total // N_Z

    def body(x_hbm, out_ref, xv, sbuf, copy_sems, send_sems, recv_sems):
        my_x = lax.axis_index("x")
        my_y = lax.axis_index("y")
        my_z = lax.axis_index("z")

        copies = []
        for slot, dz in enumerate((1, 2, 3, 0)):
            tz = (my_z + dz) % N_Z
            cp = pltpu.make_async_copy(
                x_hbm.at[:, pl.ds(tz * n_per, n_per)],
                xv.at[slot],
                copy_sems.at[slot],
            )
            cp.start()
            copies.append(cp)

        barrier_sem = pltpu.get_barrier_semaphore()
        for dz in range(1, N_Z):
            pl.semaphore_signal(
                barrier_sem, inc=1,
                device_id=(my_x, my_y, (my_z + dz) % N_Z),
                device_id_type=pl.DeviceIdType.MESH,
            )
        pl.semaphore_wait(barrier_sem, N_Z - 1)

        sends = []
        for slot, dz in enumerate((1, 2, 3)):
            tz = (my_z + dz) % N_Z
            copies[slot].wait()
            sbuf[slot] = xv[slot].astype(jnp.bfloat16)
            rdma = pltpu.make_async_remote_copy(
                src_ref=sbuf.at[slot],
                dst_ref=out_ref.at[pl.ds(my_z * m_per, m_per), :],
                send_sem=send_sems.at[dz],
                recv_sem=recv_sems.at[dz],
                device_id=(my_x, my_y, tz),
                device_id_type=pl.DeviceIdType.MESH,
            )
            rdma.start()
            sends.append(rdma)

        copies[3].wait()
        out_ref[pl.ds(my_z * m_per, m_per), :] = xv[3].astype(jnp.bfloat16)

        for dz in range(1, N_Z):
            sz = (my_z - dz) % N_Z
            recv = pltpu.make_async_remote_copy(
                src_ref=sbuf.at[0],
                dst_ref=out_ref.at[pl.ds(sz * m_per, m_per), :],
                send_sem=send_sems.at[0],
                recv_sem=recv_sems.at[dz],
                device_id=(my_x, my_y, sz),
                device_id_type=pl.DeviceIdType.MESH,
            )
            recv.wait_recv()

        for rdma in sends:
            rdma.wait_send()

    out_shape = jax.ShapeDtypeStruct((N_Z * m_per, n_per), jnp.bfloat16)
    return pl.pallas_call(
        body,
        out_shape=out_shape,
        in_specs=[pl.BlockSpec(memory_space=pl.ANY)],
        out_specs=pl.BlockSpec(memory_space=pltpu.VMEM),
        scratch_shapes=[
            pltpu.VMEM((N_Z, m_per, n_per), jnp.float32),
            pltpu.VMEM((3, m_per, n_per), jnp.bfloat16),
            pltpu.SemaphoreType.DMA((N_Z,)),
            pltpu.SemaphoreType.DMA((N_Z,)),
            pltpu.SemaphoreType.DMA((N_Z,)),
        ],
        compiler_params=pltpu.CompilerParams(collective_id=0),
    )(x)


# device time: 77516 ns/iter; 1.0673x vs baseline; 1.0120x over previous
import jax
import jax.numpy as jnp
from jax import lax
from jax.experimental import pallas as pl
from jax.experimental.pallas import tpu as pltpu

N_Z = 4


def kernel(x):
    m_per, n_total = x.shape
    n_per = n_total // N_Z

    def body(x_hbm, out_ref, xv, sbuf, copy_sems, send_sems, recv_sems):
        my_x = lax.axis_index("x")
        my_y = lax.axis_index("y")
        my_z = lax.axis_index("z")

        is_low = my_z < 2
        targets = [
            jnp.where(is_low, 3, 0),
            jnp.where(is_low, 2, 1),
            jnp.where(is_low, 1 - my_z, 5 - my_z),
        ]

        copies = []
        for slot, tz in enumerate(targets + [my_z]):
            cp = pltpu.make_async_copy(
                x_hbm.at[:, pl.ds(tz * n_per, n_per)],
                xv.at[slot],
                copy_sems.at[slot],
            )
            cp.start()
            copies.append(cp)

        barrier_sem = pltpu.get_barrier_semaphore()
        for dz in range(1, N_Z):
            pl.semaphore_signal(
                barrier_sem, inc=1,
                device_id=(my_x, my_y, (my_z + dz) % N_Z),
                device_id_type=pl.DeviceIdType.MESH,
            )
        pl.semaphore_wait(barrier_sem, N_Z - 1)

        sends = []
        for slot, tz in enumerate(targets):
            dz = (tz - my_z) % N_Z
            copies[slot].wait()
            sbuf[slot] = xv[slot].astype(jnp.bfloat16)
            rdma = pltpu.make_async_remote_copy(
                src_ref=sbuf.at[slot],
                dst_ref=out_ref.at[pl.ds(my_z * m_per, m_per), :],
                send_sem=send_sems.at[dz],
                recv_sem=recv_sems.at[dz],
                device_id=(my_x, my_y, tz),
                device_id_type=pl.DeviceIdType.MESH,
            )
            rdma.start()
            sends.append(rdma)

        copies[3].wait()
        out_ref[pl.ds(my_z * m_per, m_per), :] = xv[3].astype(jnp.bfloat16)

        for dz in range(1, N_Z):
            sz = (my_z - dz) % N_Z
            recv = pltpu.make_async_remote_copy(
                src_ref=sbuf.at[0],
                dst_ref=out_ref.at[pl.ds(sz * m_per, m_per), :],
                send_sem=send_sems.at[0],
                recv_sem=recv_sems.at[dz],
                device_id=(my_x, my_y, sz),
                device_id_type=pl.DeviceIdType.MESH,
            )
            recv.wait_recv()

        for rdma in sends:
            rdma.wait_send()

    out_shape = jax.ShapeDtypeStruct((N_Z * m_per, n_per), jnp.bfloat16)
    return pl.pallas_call(
        body,
        out_shape=out_shape,
        in_specs=[pl.BlockSpec(memory_space=pl.ANY)],
        out_specs=pl.BlockSpec(memory_space=pltpu.VMEM),
        scratch_shapes=[
            pltpu.VMEM((N_Z, m_per, n_per), jnp.float32),
            pltpu.VMEM((3, m_per, n_per), jnp.bfloat16),
            pltpu.SemaphoreType.DMA((N_Z,)),
            pltpu.SemaphoreType.DMA((N_Z,)),
            pltpu.SemaphoreType.DMA((N_Z,)),
        ],
        compiler_params=pltpu.CompilerParams(collective_id=0),
    )(x)


# device time: 77501 ns/iter; 1.0675x vs baseline; 1.0002x over previous
import jax
import jax.numpy as jnp
from jax import lax
from jax.experimental import pallas as pl
from jax.experimental.pallas import tpu as pltpu

N_Z = 4


def kernel(x):
    m_per, n_total = x.shape
    n_per = n_total // N_Z

    def body(x_hbm, out_ref, xv, sbuf, copy_sems, send_sems, recv_sems):
        my_x = lax.axis_index("x")
        my_y = lax.axis_index("y")
        my_z = lax.axis_index("z")

        is_low = my_z < 2
        targets = [
            jnp.where(is_low, 3, 0),
            jnp.where(is_low, 2, 1),
            jnp.where(is_low, 1 - my_z, 5 - my_z),
        ]

        copies = []
        for slot, tz in enumerate(targets + [my_z]):
            cp = pltpu.make_async_copy(
                x_hbm.at[:, pl.ds(tz * n_per, n_per)],
                xv.at[slot],
                copy_sems.at[slot],
            )
            cp.start()
            copies.append(cp)

        barrier_sem = pltpu.get_barrier_semaphore()
        for dz in range(1, N_Z):
            pl.semaphore_signal(
                barrier_sem, inc=1,
                device_id=(my_x, my_y, (my_z + dz) % N_Z),
                device_id_type=pl.DeviceIdType.MESH,
            )
        pl.semaphore_wait(barrier_sem, N_Z - 1)

        sends = []
        for slot, tz in enumerate(targets):
            dz = (tz - my_z) % N_Z
            copies[slot].wait()
            sbuf[slot] = xv[slot].astype(jnp.bfloat16)
            rdma = pltpu.make_async_remote_copy(
                src_ref=sbuf.at[slot],
                dst_ref=out_ref.at[pl.ds(my_z * m_per, m_per), :],
                send_sem=send_sems.at[dz],
                recv_sem=recv_sems.at[dz],
                device_id=(my_x, my_y, tz),
                device_id_type=pl.DeviceIdType.MESH,
            )
            rdma.start()
            sends.append(rdma)

        copies[3].wait()
        sbuf[3] = xv[3].astype(jnp.bfloat16)
        local_cp = pltpu.make_async_copy(
            sbuf.at[3],
            out_ref.at[pl.ds(my_z * m_per, m_per), :],
            copy_sems.at[3],
        )
        local_cp.start()
        local_cp.wait()

        for dz in range(1, N_Z):
            sz = (my_z - dz) % N_Z
            recv = pltpu.make_async_remote_copy(
                src_ref=sbuf.at[0],
                dst_ref=out_ref.at[pl.ds(sz * m_per, m_per), :],
                send_sem=send_sems.at[0],
                recv_sem=recv_sems.at[dz],
                device_id=(my_x, my_y, sz),
                device_id_type=pl.DeviceIdType.MESH,
            )
            recv.wait_recv()

        for rdma in sends:
            rdma.wait_send()

    out_shape = jax.ShapeDtypeStruct((N_Z * m_per, n_per), jnp.bfloat16)
    return pl.pallas_call(
        body,
        out_shape=out_shape,
        in_specs=[pl.BlockSpec(memory_space=pl.ANY)],
        out_specs=pl.BlockSpec(memory_space=pl.ANY),
        scratch_shapes=[
            pltpu.VMEM((N_Z, m_per, n_per), jnp.float32),
            pltpu.VMEM((N_Z, m_per, n_per), jnp.bfloat16),
            pltpu.SemaphoreType.DMA((N_Z,)),
            pltpu.SemaphoreType.DMA((N_Z,)),
            pltpu.SemaphoreType.DMA((N_Z,)),
        ],
        compiler_params=pltpu.CompilerParams(collective_id=0),
    )(x)
